# baseline (speedup 1.0000x reference)
import functools

import jax
import jax.numpy as jnp
from jax import lax
from jax.experimental import pallas as pl
from jax.experimental.pallas import tpu as pltpu
from jax.experimental.pallas import tpu_sc as plsc

N = 100000
E = 1600000
B = 1024

NC = 2
NS = 16
NW = NC * NS

LANE = 128
E_ROWS_V = E // LANE
E_ROWS = 12544
EROWS_PER_W = E_ROWS // NW
EROWS_PER_T = E_ROWS // NS

NB_ROWS = 1024
N_PAD = NB_ROWS * LANE
N_ACC = 100096
B_ACC = 1152
NPT = N_ACC // NS
BPT = B_ACC // NS

DOFF = 102000

F32 = jnp.float32
I32 = jnp.int32

_MESH = dict(core_axis_name="c", subcore_axis_name="s")
_NOTILE = pltpu.CompilerParams(use_tc_tiling_on_sc=False)


def _worker_ids():
    cid = lax.axis_index("c")
    sid = lax.axis_index("s")
    return cid, sid, cid * NS + sid


def _fill(buf, rows, cols, val):
    v = jnp.full((16,), val, F32)
    for r in range(rows):
        for c0 in range(0, cols, 16):
            buf[r, pl.ds(c0, 16)] = v


def _zero_acc_16(zsrc, zrows, acc, base):
    def zstep(t, carry):
        pltpu.sync_copy(zsrc.at[pl.ds(0, zrows)],
                        acc.at[pl.ds(base + t * zrows, zrows)])
        return carry

    lax.fori_loop(0, NPT // zrows, zstep, 0)
    rem = NPT % zrows
    if rem:
        pltpu.sync_copy(zsrc.at[pl.ds(0, rem)],
                        acc.at[pl.ds(base + (NPT // zrows) * zrows, rem)])


def _sc_deg_norm(edge2d, x16):
    KD = 8
    CH = 125
    NPC = N // NC
    NPS = NPC // NS
    NCH = NPS // CH

    @functools.partial(
        pl.kernel,
        out_type=[jax.ShapeDtypeStruct((2 * N, 16), F32)],
        mesh=plsc.VectorSubcoreMesh(**_MESH),
        compiler_params=_NOTILE,
        scratch_types=[
            pltpu.VMEM((KD, LANE), I32),
            pltpu.VMEM((LANE, 16), F32),
            pltpu.VMEM((LANE, 16), F32),
            pltpu.VMEM((CH, 16), F32),
            pltpu.VMEM((CH, 16), F32),
            pltpu.VMEM((CH, 16), F32),
            pltpu.VMEM((CH, 16), F32),
            pltpu.VMEM_SHARED((N_ACC, 16), F32),
            pltpu.SemaphoreType.DMA,
        ],
    )
    def body(edge_h, x_h, pdis_o, didx, ones_v, zbuf, degb, xb, pb, db,
             dacc, sem):
        cid, sid, w = _worker_ids()
        _fill(ones_v, LANE, 16, 1.0)
        _fill(zbuf, LANE, 16, 0.0)
        _zero_acc_16(zbuf, LANE, dacc, sid * NPT)
        plsc.subcore_barrier()

        def deg_step(t, carry):
            rb = sid * EROWS_PER_T + t * KD
            rbs = jnp.minimum(rb, E_ROWS_V - KD)
            delta = rb - rbs
            pltpu.sync_copy(edge_h.at[pl.ds(E_ROWS_V + rbs, KD)], didx)
            for r in range(KD):
                m = jnp.where(delta <= r, 1, 0).astype(I32)
                junk = (1 - m) * N
                for c0 in range(0, LANE, 16):
                    didx[r, pl.ds(c0, 16)] = didx[r, pl.ds(c0, 16)] * m + junk
            ds = [pltpu.async_copy(ones_v, dacc.at[didx.at[j]], sem, add=True)
                  for j in range(KD)]
            for d in ds:
                d.wait()
            return carry

        lax.fori_loop(0, EROWS_PER_T // KD, deg_step, 0)
        plsc.subcore_barrier()

        base = cid * NPC + sid * NPS

        def norm_step(t, carry):
            nb = base + t * CH
            pltpu.sync_copy(dacc.at[pl.ds(nb, CH)], degb)
            pltpu.sync_copy(x_h.at[pl.ds(nb, CH)], xb)
            for r in range(CH):
                d = degb[r, pl.ds(0, 16)] + 1.0
                xi = lax.bitcast_convert_type(d, I32)
                yi = jnp.int32(0x5F3759DF) - (xi >> 1)
                y = lax.bitcast_convert_type(yi, F32)
                for _ in range(3):
                    y = y * (1.5 - (0.5 * d) * (y * y))
                db[r, pl.ds(0, 16)] = y
                pb[r, pl.ds(0, 16)] = y * xb[r, pl.ds(0, 16)]
            pltpu.sync_copy(pb, pdis_o.at[pl.ds(nb, CH)])
            pltpu.sync_copy(db, pdis_o.at[pl.ds(N + nb, CH)])
            return carry

        lax.fori_loop(0, NCH, norm_step, 0)

    return body(edge2d, x16)[0]


def _sc_edge_agg(tab, edge2d, split_edges):
    k = 7
    if split_edges:
        rows_per, off = EROWS_PER_W, DOFF
    else:
        rows_per, off = EROWS_PER_T, DOFF

    @functools.partial(
        pl.kernel,
        out_type=[jax.ShapeDtypeStruct((off + N_ACC, 16), F32)],
        mesh=plsc.VectorSubcoreMesh(**_MESH),
        compiler_params=_NOTILE,
        scratch_types=[
            pltpu.VMEM((k, LANE), I32),
            pltpu.VMEM((k, LANE), I32),
            pltpu.VMEM((k * LANE, 16), F32),
            pltpu.VMEM_SHARED((N_ACC, 16), F32),
            pltpu.SemaphoreType.DMA,
            pltpu.SemaphoreType.DMA,
        ],
    )
    def body(tab_h, edge_h, out_h, sidx, didx, rows, acc, gsem, ssem):
        cid, sid, w = _worker_ids()
        _fill(rows, LANE, 16, 0.0)
        _zero_acc_16(rows, LANE, acc, sid * NPT)
        plsc.subcore_barrier()

        base = w * rows_per if split_edges else sid * rows_per

        def step(t, carry):
            rb = base + t * k
            rbs = jnp.minimum(rb, E_ROWS_V - k)
            delta = rb - rbs
            pltpu.sync_copy(edge_h.at[pl.ds(rbs, k)], sidx)
            pltpu.sync_copy(edge_h.at[pl.ds(E_ROWS_V + rbs, k)], didx)
            shift = cid * N if not split_edges else 0
            for r in range(k):
                m = jnp.where(delta <= r, 1, 0).astype(I32)
                junk = (1 - m) * N
                for c0 in range(0, LANE, 16):
                    sidx[r, pl.ds(c0, 16)] = sidx[r, pl.ds(c0, 16)] * m + shift
                    didx[r, pl.ds(c0, 16)] = didx[r, pl.ds(c0, 16)] * m + junk
            gds = [pltpu.async_copy(tab_h.at[sidx.at[j]],
                                    rows.at[pl.ds(j * LANE, LANE)], gsem)
                   for j in range(k)]
            for d in gds:
                d.wait()
            sds = [pltpu.async_copy(rows.at[pl.ds(j * LANE, LANE)],
                                    acc.at[didx.at[j]], ssem, add=True)
                   for j in range(k)]
            for d in sds:
                d.wait()
            return carry

        lax.fori_loop(0, rows_per // k, step, 0)
        plsc.subcore_barrier()
        pltpu.sync_copy(acc.at[pl.ds(sid * NPT, NPT)],
                        out_h.at[pl.ds(cid * off + sid * NPT, NPT)])

    return body(tab, edge2d)[0]


def _sc_pool(h2p, batch2d):
    KP = 8
    npw = N_PAD // NW

    @functools.partial(
        pl.kernel,
        out_type=[jax.ShapeDtypeStruct((2 * B_ACC, 32), F32),
                  jax.ShapeDtypeStruct((2 * B_ACC, 16), F32)],
        mesh=plsc.VectorSubcoreMesh(**_MESH),
        compiler_params=_NOTILE,
        scratch_types=[
            pltpu.VMEM((KP, LANE), I32),
            pltpu.VMEM((KP * LANE, 32), F32),
            pltpu.VMEM((LANE, 16), F32),
            pltpu.VMEM((BPT, 32), F32),
            pltpu.VMEM((BPT, 16), F32),
            pltpu.VMEM_SHARED((B_ACC, 32), F32),
            pltpu.VMEM_SHARED((B_ACC, 16), F32),
            pltpu.SemaphoreType.DMA,
        ],
    )
    def body(h2_h, bat_h, poolo, cnto,
             bidx, rows, ones_v, zbuf, zbuf16, acc, cacc, sem):
        cid, sid, w = _worker_ids()
        _fill(ones_v, LANE, 16, 1.0)
        _fill(zbuf, BPT, 32, 0.0)
        _fill(zbuf16, BPT, 16, 0.0)
        pltpu.sync_copy(zbuf, acc.at[pl.ds(sid * BPT, BPT)])
        pltpu.sync_copy(zbuf16, cacc.at[pl.ds(sid * BPT, BPT)])
        plsc.subcore_barrier()

        def step(t, carry):
            pltpu.sync_copy(bat_h.at[pl.ds(w * (npw // LANE) + t * KP, KP)], bidx)
            pltpu.sync_copy(h2_h.at[pl.ds(w * npw + t * KP * LANE, KP * LANE)],
                            rows)
            ds = [pltpu.async_copy(rows.at[pl.ds(j * LANE, LANE)],
                                   acc.at[bidx.at[j]], sem, add=True)
                  for j in range(KP)]
            ds += [pltpu.async_copy(ones_v, cacc.at[bidx.at[j]], sem, add=True)
                   for j in range(KP)]
            for d in ds:
                d.wait()
            return carry

        lax.fori_loop(0, npw // (KP * LANE), step, 0)
        plsc.subcore_barrier()
        pltpu.sync_copy(acc.at[pl.ds(sid * BPT, BPT)],
                        poolo.at[pl.ds(cid * B_ACC + sid * BPT, BPT)])
        pltpu.sync_copy(cacc.at[pl.ds(sid * BPT, BPT)],
                        cnto.at[pl.ds(cid * B_ACC + sid * BPT, BPT)])

    return body(h2p, batch2d)


def _tc_layer1(qo, x16, pdis, W1p, W1b, b1r, W2):
    BN = 2000

    def body(q0r, q1r, xr, disr, w1r, w1br, b1rr, w2r, r_o, g_o):
        dis = disr[:, :1]
        xw = jnp.dot(xr[...], w1r[...], preferred_element_type=F32)
        qt = jnp.dot(dis * (q0r[...] + q1r[...]), w1br[...],
                     preferred_element_type=F32,
                     precision=lax.Precision.HIGHEST)
        h1 = jnp.maximum(qt + (dis * dis) * xw + b1rr[...], 0.0)
        g = jnp.dot(h1, w2r[...], preferred_element_type=F32)
        r = dis * g
        r_o[0] = r[:, :16]
        r_o[1] = r[:, 16:]
        g_o[...] = g

    return pl.pallas_call(
        body,
        grid=(N // BN,),
        in_specs=[pl.BlockSpec((BN, 16), lambda i: (i, 0)),
                  pl.BlockSpec((BN, 16), lambda i: (i + DOFF // BN, 0)),
                  pl.BlockSpec((BN, 16), lambda i: (i, 0)),
                  pl.BlockSpec((BN, 16), lambda i: (i + N // BN, 0)),
                  pl.BlockSpec((16, 64), lambda i: (0, 0)),
                  pl.BlockSpec((16, 64), lambda i: (0, 0)),
                  pl.BlockSpec((1, 64), lambda i: (0, 0)),
                  pl.BlockSpec((64, 32), lambda i: (0, 0))],
        out_specs=[pl.BlockSpec((2, BN, 16), lambda i: (0, i, 0)),
                   pl.BlockSpec((BN, 32), lambda i: (i, 0))],
        out_shape=[jax.ShapeDtypeStruct((2, N, 16), F32),
                   jax.ShapeDtypeStruct((N, 32), F32)],
    )(qo, qo, x16, pdis, W1p, W1b, b1r, W2)


def _tc_layer2(so, g, pdis, b2r):
    BN = 2000

    def body(s0r, s1r, gr, disr, b2rr, h2_o):
        dis = disr[:, :1]
        s = jnp.concatenate([s0r[...], s1r[...]], axis=1)
        h2_o[...] = jnp.maximum(dis * s + (dis * dis) * gr[...] + b2rr[...], 0.0)

    return pl.pallas_call(
        body,
        grid=(N // BN,),
        in_specs=[pl.BlockSpec((BN, 16), lambda i: (i, 0)),
                  pl.BlockSpec((BN, 16), lambda i: (i + DOFF // BN, 0)),
                  pl.BlockSpec((BN, 32), lambda i: (i, 0)),
                  pl.BlockSpec((BN, 16), lambda i: (i + N // BN, 0)),
                  pl.BlockSpec((1, 32), lambda i: (0, 0))],
        out_specs=pl.BlockSpec((BN, 32), lambda i: (i, 0)),
        out_shape=jax.ShapeDtypeStruct((N_PAD, 32), F32),
    )(so, so, g, pdis, b2r)


def _tc_head(poolo, cnto, Wf1, bf1r, Wf2, bf2r, Wf3, bf3r):
    def body(pr, cr, w1r, b1r, w2r, b2r, w3r, b3r, o):
        cnt = jnp.maximum(cr[:B, :1] + cr[B_ACC:B_ACC + B, :1], 1.0)
        pool = (pr[:B, :] + pr[B_ACC:B_ACC + B, :]) / cnt
        a = jnp.maximum(jnp.dot(pool, w1r[...], preferred_element_type=F32)
                        + b1r[...], 0.0)
        a = jnp.maximum(jnp.dot(a, w2r[...], preferred_element_type=F32)
                        + b2r[...], 0.0)
        o[...] = jnp.dot(a, w3r[...], preferred_element_type=F32) + b3r[...]

    return pl.pallas_call(
        body,
        out_shape=jax.ShapeDtypeStruct((B, 1), F32),
    )(poolo, cnto, Wf1, bf1r, Wf2, bf2r, Wf3, bf3r)


def kernel(x, edge_index, batch, static_feature,
           W1, b1, W2, b2, Wf1, bf1, Wf2, bf2, Wf3, bf3):
    del static_feature

    x16 = jnp.pad(x, ((0, 0), (0, 7)))
    x16b = lax.reduce_precision(x16, 8, 7)
    W1p = jnp.pad(W1, ((0, 7), (0, 0)))
    W1b = lax.reduce_precision(W1p, 8, 7)
    edge2d = edge_index.reshape(2 * E_ROWS_V, LANE)
    batch2d = jnp.concatenate(
        [batch, jnp.full((N_PAD - N,), B, I32)]).reshape(NB_ROWS, LANE)
    b1r = b1.reshape(1, 64)
    b2r = b2.reshape(1, 32)
    bf1r, bf2r, bf3r = bf1.reshape(1, 16), bf2.reshape(1, 8), bf3.reshape(1, 1)

    pdis = _sc_deg_norm(edge2d, x16b)

    qo = _sc_edge_agg(pdis, edge2d, split_edges=True)
    rstk, g = _tc_layer1(qo, x16, pdis, W1p, W1b, b1r, W2)

    so = _sc_edge_agg(rstk.reshape(2 * N, 16), edge2d, split_edges=False)
    h2p = _tc_layer2(so, g, pdis, b2r)

    poolo, cnto = _sc_pool(h2p, batch2d)
    return _tc_head(poolo, cnto, Wf1, bf1r, Wf2, bf2r, Wf3, bf3r)


# speedup vs baseline: 1.0067x; 1.0067x over previous
import functools

import jax
import jax.numpy as jnp
from jax import lax
from jax.experimental import pallas as pl
from jax.experimental.pallas import tpu as pltpu
from jax.experimental.pallas import tpu_sc as plsc

N = 100000
E = 1600000
B = 1024

NC = 2
NS = 16
NW = NC * NS

LANE = 128
E_ROWS_V = E // LANE
E_ROWS = 12544
EROWS_PER_W = E_ROWS // NW
EROWS_PER_T = E_ROWS // NS

NB_ROWS = 1024
N_PAD = NB_ROWS * LANE
N_ACC = 100096
B_ACC = 1152
NPT = N_ACC // NS
BPT = B_ACC // NS

DOFF = 102000

F32 = jnp.float32
I32 = jnp.int32

_MESH = dict(core_axis_name="c", subcore_axis_name="s")
_NOTILE = pltpu.CompilerParams(use_tc_tiling_on_sc=False)


def _worker_ids():
    cid = lax.axis_index("c")
    sid = lax.axis_index("s")
    return cid, sid, cid * NS + sid


def _fill(buf, rows, cols, val):
    v = jnp.full((16,), val, F32)
    for r in range(rows):
        for c0 in range(0, cols, 16):
            buf[r, pl.ds(c0, 16)] = v


def _zero_acc_16(zsrc, zrows, acc, base):
    def zstep(t, carry):
        pltpu.sync_copy(zsrc.at[pl.ds(0, zrows)],
                        acc.at[pl.ds(base + t * zrows, zrows)])
        return carry

    lax.fori_loop(0, NPT // zrows, zstep, 0)
    rem = NPT % zrows
    if rem:
        pltpu.sync_copy(zsrc.at[pl.ds(0, rem)],
                        acc.at[pl.ds(base + (NPT // zrows) * zrows, rem)])


def _sc_deg_norm(edge2d, x16):
    KD = 8
    CH = 125
    NPC = N // NC
    NPS = NPC // NS
    NCH = NPS // CH

    @functools.partial(
        pl.kernel,
        out_type=[jax.ShapeDtypeStruct((2 * N, 16), F32)],
        mesh=plsc.VectorSubcoreMesh(**_MESH),
        compiler_params=_NOTILE,
        scratch_types=[
            pltpu.VMEM((KD, LANE), I32),
            pltpu.VMEM((LANE, 16), F32),
            pltpu.VMEM((LANE, 16), F32),
            pltpu.VMEM((CH, 16), F32),
            pltpu.VMEM((CH, 16), F32),
            pltpu.VMEM((CH, 16), F32),
            pltpu.VMEM((CH, 16), F32),
            pltpu.VMEM_SHARED((N_ACC, 16), F32),
            pltpu.SemaphoreType.DMA,
        ],
    )
    def body(edge_h, x_h, pdis_o, didx, ones_v, zbuf, degb, xb, pb, db,
             dacc, sem):
        cid, sid, w = _worker_ids()
        _fill(ones_v, LANE, 16, 1.0)
        _fill(zbuf, LANE, 16, 0.0)
        _zero_acc_16(zbuf, LANE, dacc, sid * NPT)
        plsc.subcore_barrier()

        def deg_step(t, carry):
            rb = sid * EROWS_PER_T + t * KD
            rbs = jnp.minimum(rb, E_ROWS_V - KD)
            delta = rb - rbs
            pltpu.sync_copy(edge_h.at[pl.ds(E_ROWS_V + rbs, KD)], didx)
            for r in range(KD):
                m = jnp.where(delta <= r, 1, 0).astype(I32)
                junk = (1 - m) * N
                for c0 in range(0, LANE, 16):
                    didx[r, pl.ds(c0, 16)] = didx[r, pl.ds(c0, 16)] * m + junk
            ds = [pltpu.async_copy(ones_v, dacc.at[didx.at[j]], sem, add=True)
                  for j in range(KD)]
            for d in ds:
                d.wait()
            return carry

        lax.fori_loop(0, EROWS_PER_T // KD, deg_step, 0)
        plsc.subcore_barrier()

        base = cid * NPC + sid * NPS

        def norm_step(t, carry):
            nb = base + t * CH
            pltpu.sync_copy(dacc.at[pl.ds(nb, CH)], degb)
            pltpu.sync_copy(x_h.at[pl.ds(nb, CH)], xb)
            for r in range(CH):
                d = degb[r, pl.ds(0, 16)] + 1.0
                xi = lax.bitcast_convert_type(d, I32)
                yi = jnp.int32(0x5F3759DF) - (xi >> 1)
                y = lax.bitcast_convert_type(yi, F32)
                for _ in range(3):
                    y = y * (1.5 - (0.5 * d) * (y * y))
                db[r, pl.ds(0, 16)] = y
                vi = lax.bitcast_convert_type(xb[r, pl.ds(0, 16)], I32)
                vi = (vi + jnp.int32(0x7FFF) + ((vi >> 16) & 1)) & jnp.int32(-65536)
                pb[r, pl.ds(0, 16)] = y * lax.bitcast_convert_type(vi, F32)
            pltpu.sync_copy(pb, pdis_o.at[pl.ds(nb, CH)])
            pltpu.sync_copy(db, pdis_o.at[pl.ds(N + nb, CH)])
            return carry

        lax.fori_loop(0, NCH, norm_step, 0)

    return body(edge2d, x16)[0]


def _sc_edge_agg(tab, edge2d, split_edges):
    k = 7
    if split_edges:
        rows_per, off = EROWS_PER_W, DOFF
    else:
        rows_per, off = EROWS_PER_T, DOFF

    @functools.partial(
        pl.kernel,
        out_type=[jax.ShapeDtypeStruct((off + N_ACC, 16), F32)],
        mesh=plsc.VectorSubcoreMesh(**_MESH),
        compiler_params=_NOTILE,
        scratch_types=[
            pltpu.VMEM((k, LANE), I32),
            pltpu.VMEM((k, LANE), I32),
            pltpu.VMEM((k * LANE, 16), F32),
            pltpu.VMEM_SHARED((N_ACC, 16), F32),
            pltpu.SemaphoreType.DMA,
            pltpu.SemaphoreType.DMA,
        ],
    )
    def body(tab_h, edge_h, out_h, sidx, didx, rows, acc, gsem, ssem):
        cid, sid, w = _worker_ids()
        _fill(rows, LANE, 16, 0.0)
        _zero_acc_16(rows, LANE, acc, sid * NPT)
        plsc.subcore_barrier()

        base = w * rows_per if split_edges else sid * rows_per

        def step(t, carry):
            rb = base + t * k
            rbs = jnp.minimum(rb, E_ROWS_V - k)
            delta = rb - rbs
            pltpu.sync_copy(edge_h.at[pl.ds(rbs, k)], sidx)
            pltpu.sync_copy(edge_h.at[pl.ds(E_ROWS_V + rbs, k)], didx)
            shift = cid * N if not split_edges else 0
            for r in range(k):
                m = jnp.where(delta <= r, 1, 0).astype(I32)
                junk = (1 - m) * N
                for c0 in range(0, LANE, 16):
                    sidx[r, pl.ds(c0, 16)] = sidx[r, pl.ds(c0, 16)] * m + shift
                    didx[r, pl.ds(c0, 16)] = didx[r, pl.ds(c0, 16)] * m + junk
            gds = [pltpu.async_copy(tab_h.at[sidx.at[j]],
                                    rows.at[pl.ds(j * LANE, LANE)], gsem)
                   for j in range(k)]
            for d in gds:
                d.wait()
            sds = [pltpu.async_copy(rows.at[pl.ds(j * LANE, LANE)],
                                    acc.at[didx.at[j]], ssem, add=True)
                   for j in range(k)]
            for d in sds:
                d.wait()
            return carry

        lax.fori_loop(0, rows_per // k, step, 0)
        plsc.subcore_barrier()
        pltpu.sync_copy(acc.at[pl.ds(sid * NPT, NPT)],
                        out_h.at[pl.ds(cid * off + sid * NPT, NPT)])

    return body(tab, edge2d)[0]


def _sc_pool(h2p, batch2d):
    KP = 8
    npw = N_PAD // NW

    @functools.partial(
        pl.kernel,
        out_type=[jax.ShapeDtypeStruct((2 * B_ACC, 32), F32),
                  jax.ShapeDtypeStruct((2 * B_ACC, 16), F32)],
        mesh=plsc.VectorSubcoreMesh(**_MESH),
        compiler_params=_NOTILE,
        scratch_types=[
            pltpu.VMEM((KP, LANE), I32),
            pltpu.VMEM((KP * LANE, 32), F32),
            pltpu.VMEM((LANE, 16), F32),
            pltpu.VMEM((BPT, 32), F32),
            pltpu.VMEM((BPT, 16), F32),
            pltpu.VMEM_SHARED((B_ACC, 32), F32),
            pltpu.VMEM_SHARED((B_ACC, 16), F32),
            pltpu.SemaphoreType.DMA,
        ],
    )
    def body(h2_h, bat_h, poolo, cnto,
             bidx, rows, ones_v, zbuf, zbuf16, acc, cacc, sem):
        cid, sid, w = _worker_ids()
        _fill(ones_v, LANE, 16, 1.0)
        _fill(zbuf, BPT, 32, 0.0)
        _fill(zbuf16, BPT, 16, 0.0)
        pltpu.sync_copy(zbuf, acc.at[pl.ds(sid * BPT, BPT)])
        pltpu.sync_copy(zbuf16, cacc.at[pl.ds(sid * BPT, BPT)])
        plsc.subcore_barrier()

        def step(t, carry):
            pltpu.sync_copy(bat_h.at[pl.ds(w * (npw // LANE) + t * KP, KP)], bidx)
            pltpu.sync_copy(h2_h.at[pl.ds(w * npw + t * KP * LANE, KP * LANE)],
                            rows)
            ds = [pltpu.async_copy(rows.at[pl.ds(j * LANE, LANE)],
                                   acc.at[bidx.at[j]], sem, add=True)
                  for j in range(KP)]
            ds += [pltpu.async_copy(ones_v, cacc.at[bidx.at[j]], sem, add=True)
                   for j in range(KP)]
            for d in ds:
                d.wait()
            return carry

        lax.fori_loop(0, npw // (KP * LANE), step, 0)
        plsc.subcore_barrier()
        pltpu.sync_copy(acc.at[pl.ds(sid * BPT, BPT)],
                        poolo.at[pl.ds(cid * B_ACC + sid * BPT, BPT)])
        pltpu.sync_copy(cacc.at[pl.ds(sid * BPT, BPT)],
                        cnto.at[pl.ds(cid * B_ACC + sid * BPT, BPT)])

    return body(h2p, batch2d)


def _tc_layer1(qo, x16, pdis, W1p, W1b, b1r, W2):
    BN = 2000

    def body(q0r, q1r, xr, disr, w1r, w1br, b1rr, w2r, r_o, g_o):
        dis = disr[:, :1]
        xw = jnp.dot(xr[...], w1r[...], preferred_element_type=F32)
        qt = jnp.dot(dis * (q0r[...] + q1r[...]), w1br[...],
                     preferred_element_type=F32,
                     precision=lax.Precision.HIGHEST)
        h1 = jnp.maximum(qt + (dis * dis) * xw + b1rr[...], 0.0)
        g = jnp.dot(h1, w2r[...], preferred_element_type=F32)
        r = dis * g
        r_o[0] = r[:, :16]
        r_o[1] = r[:, 16:]
        g_o[...] = g

    return pl.pallas_call(
        body,
        grid=(N // BN,),
        in_specs=[pl.BlockSpec((BN, 16), lambda i: (i, 0)),
                  pl.BlockSpec((BN, 16), lambda i: (i + DOFF // BN, 0)),
                  pl.BlockSpec((BN, 16), lambda i: (i, 0)),
                  pl.BlockSpec((BN, 16), lambda i: (i + N // BN, 0)),
                  pl.BlockSpec((16, 64), lambda i: (0, 0)),
                  pl.BlockSpec((16, 64), lambda i: (0, 0)),
                  pl.BlockSpec((1, 64), lambda i: (0, 0)),
                  pl.BlockSpec((64, 32), lambda i: (0, 0))],
        out_specs=[pl.BlockSpec((2, BN, 16), lambda i: (0, i, 0)),
                   pl.BlockSpec((BN, 32), lambda i: (i, 0))],
        out_shape=[jax.ShapeDtypeStruct((2, N, 16), F32),
                   jax.ShapeDtypeStruct((N, 32), F32)],
    )(qo, qo, x16, pdis, W1p, W1b, b1r, W2)


def _tc_layer2(so, g, pdis, b2r):
    BN = 2000

    def body(s0r, s1r, gr, disr, b2rr, h2_o):
        dis = disr[:, :1]
        s = jnp.concatenate([s0r[...], s1r[...]], axis=1)
        h2_o[...] = jnp.maximum(dis * s + (dis * dis) * gr[...] + b2rr[...], 0.0)

    return pl.pallas_call(
        body,
        grid=(N // BN,),
        in_specs=[pl.BlockSpec((BN, 16), lambda i: (i, 0)),
                  pl.BlockSpec((BN, 16), lambda i: (i + DOFF // BN, 0)),
                  pl.BlockSpec((BN, 32), lambda i: (i, 0)),
                  pl.BlockSpec((BN, 16), lambda i: (i + N // BN, 0)),
                  pl.BlockSpec((1, 32), lambda i: (0, 0))],
        out_specs=pl.BlockSpec((BN, 32), lambda i: (i, 0)),
        out_shape=jax.ShapeDtypeStruct((N_PAD, 32), F32),
    )(so, so, g, pdis, b2r)


def _tc_head(poolo, cnto, Wf1, bf1r, Wf2, bf2r, Wf3, bf3r):
    def body(pr, cr, w1r, b1r, w2r, b2r, w3r, b3r, o):
        cnt = jnp.maximum(cr[:B, :1] + cr[B_ACC:B_ACC + B, :1], 1.0)
        pool = (pr[:B, :] + pr[B_ACC:B_ACC + B, :]) / cnt
        a = jnp.maximum(jnp.dot(pool, w1r[...], preferred_element_type=F32)
                        + b1r[...], 0.0)
        a = jnp.maximum(jnp.dot(a, w2r[...], preferred_element_type=F32)
                        + b2r[...], 0.0)
        o[...] = jnp.dot(a, w3r[...], preferred_element_type=F32) + b3r[...]

    return pl.pallas_call(
        body,
        out_shape=jax.ShapeDtypeStruct((B, 1), F32),
    )(poolo, cnto, Wf1, bf1r, Wf2, bf2r, Wf3, bf3r)


def kernel(x, edge_index, batch, static_feature,
           W1, b1, W2, b2, Wf1, bf1, Wf2, bf2, Wf3, bf3):
    del static_feature

    x16 = jnp.pad(x, ((0, 0), (0, 7)))
    W1p = jnp.pad(W1, ((0, 7), (0, 0)))
    W1b = lax.reduce_precision(W1p, 8, 7)
    edge2d = edge_index.reshape(2 * E_ROWS_V, LANE)
    batch2d = jnp.concatenate(
        [batch, jnp.full((N_PAD - N,), B, I32)]).reshape(NB_ROWS, LANE)
    b1r = b1.reshape(1, 64)
    b2r = b2.reshape(1, 32)
    bf1r, bf2r, bf3r = bf1.reshape(1, 16), bf2.reshape(1, 8), bf3.reshape(1, 1)

    pdis = _sc_deg_norm(edge2d, x16)

    qo = _sc_edge_agg(pdis, edge2d, split_edges=True)
    rstk, g = _tc_layer1(qo, x16, pdis, W1p, W1b, b1r, W2)

    so = _sc_edge_agg(rstk.reshape(2 * N, 16), edge2d, split_edges=False)
    h2p = _tc_layer2(so, g, pdis, b2r)

    poolo, cnto = _sc_pool(h2p, batch2d)
    return _tc_head(poolo, cnto, Wf1, bf1r, Wf2, bf2r, Wf3, bf3r)
